# Initial kernel scaffold; baseline (speedup 1.0000x reference)
import jax
import jax.numpy as jnp
from jax.experimental import pallas as pl


def kernel(x, W_enc, b_enc, W_dec, b_dec):
    raise NotImplementedError("write your pallas kernel here")


# speedup vs baseline: 20.4095x; 20.4095x over previous
import functools

import jax
import jax.numpy as jnp
from jax.experimental import pallas as pl
from jax.experimental.pallas import tpu as pltpu

N = 8192
INPUT_DIM = 1024
LATENT_DIM = 4096
K = 32
BLOCK_ROWS = 256
N_ITERS = 28


def _body(x_ref, we_ref, be_ref, wd_ref, bd_ref, recon_ref, sparse_ref):
    lat = jnp.dot(x_ref[:], we_ref[:], preferred_element_type=jnp.float32)
    lat = lat + be_ref[:]

    lo = jnp.min(lat, axis=1, keepdims=True)
    hi = jnp.max(lat, axis=1, keepdims=True)

    def step(_, carry):
        lo, hi = carry
        mid = 0.5 * (lo + hi)
        cnt = jnp.sum((lat >= mid).astype(jnp.float32), axis=1, keepdims=True)
        ge = cnt >= K
        return jnp.where(ge, mid, lo), jnp.where(ge, hi, mid)

    lo, hi = jax.lax.fori_loop(0, N_ITERS, step, (lo, hi))

    sparse = jnp.where(lat >= lo, lat, 0.0)
    sparse_ref[:] = sparse
    recon = jnp.dot(sparse, wd_ref[:], preferred_element_type=jnp.float32)
    recon_ref[:] = recon + bd_ref[:]


@jax.jit
def kernel(x, W_enc, b_enc, W_dec, b_dec):
    grid = (N // BLOCK_ROWS,)
    recon, sparse = pl.pallas_call(
        _body,
        grid=grid,
        in_specs=[
            pl.BlockSpec((BLOCK_ROWS, INPUT_DIM), lambda i: (i, 0)),
            pl.BlockSpec((INPUT_DIM, LATENT_DIM), lambda i: (0, 0)),
            pl.BlockSpec((1, LATENT_DIM), lambda i: (0, 0)),
            pl.BlockSpec((LATENT_DIM, INPUT_DIM), lambda i: (0, 0)),
            pl.BlockSpec((1, INPUT_DIM), lambda i: (0, 0)),
        ],
        out_specs=[
            pl.BlockSpec((BLOCK_ROWS, INPUT_DIM), lambda i: (i, 0)),
            pl.BlockSpec((BLOCK_ROWS, LATENT_DIM), lambda i: (i, 0)),
        ],
        out_shape=[
            jax.ShapeDtypeStruct((N, INPUT_DIM), jnp.float32),
            jax.ShapeDtypeStruct((N, LATENT_DIM), jnp.float32),
        ],
        compiler_params=pltpu.CompilerParams(
            dimension_semantics=("arbitrary",),
        ),
    )(x, W_enc, b_enc.reshape(1, LATENT_DIM), W_dec, b_dec.reshape(1, INPUT_DIM))
    return recon, sparse


# speedup vs baseline: 23.9823x; 1.1751x over previous
import functools

import jax
import jax.numpy as jnp
from jax.experimental import pallas as pl
from jax.experimental.pallas import tpu as pltpu

N = 8192
INPUT_DIM = 1024
LATENT_DIM = 4096
K = 32
BLOCK_ROWS = 256
N_ITERS = 22


def _body(x_ref, we_ref, be_ref, wd_ref, bd_ref, recon_ref, sparse_ref):
    lat = jnp.dot(x_ref[:], we_ref[:], preferred_element_type=jnp.float32)
    lat = lat + be_ref[:]

    lo = jnp.min(lat, axis=1, keepdims=True)
    hi = jnp.max(lat, axis=1, keepdims=True)

    def step(_, carry):
        lo, hi = carry
        mid = 0.5 * (lo + hi)
        cnt = jnp.sum((lat >= mid).astype(jnp.float32), axis=1, keepdims=True)
        ge = cnt >= K
        return jnp.where(ge, mid, lo), jnp.where(ge, hi, mid)

    lo, hi = jax.lax.fori_loop(0, N_ITERS, step, (lo, hi))

    sparse = jnp.where(lat >= lo, lat, 0.0)
    sparse_ref[:] = sparse
    recon = jnp.dot(sparse.astype(jnp.bfloat16), wd_ref[:],
                    preferred_element_type=jnp.float32)
    recon_ref[:] = recon + bd_ref[:]


@jax.jit
def kernel(x, W_enc, b_enc, W_dec, b_dec):
    grid = (N // BLOCK_ROWS,)
    recon, sparse = pl.pallas_call(
        _body,
        grid=grid,
        in_specs=[
            pl.BlockSpec((BLOCK_ROWS, INPUT_DIM), lambda i: (i, 0)),
            pl.BlockSpec((INPUT_DIM, LATENT_DIM), lambda i: (0, 0)),
            pl.BlockSpec((1, LATENT_DIM), lambda i: (0, 0)),
            pl.BlockSpec((LATENT_DIM, INPUT_DIM), lambda i: (0, 0)),
            pl.BlockSpec((1, INPUT_DIM), lambda i: (0, 0)),
        ],
        out_specs=[
            pl.BlockSpec((BLOCK_ROWS, INPUT_DIM), lambda i: (i, 0)),
            pl.BlockSpec((BLOCK_ROWS, LATENT_DIM), lambda i: (i, 0)),
        ],
        out_shape=[
            jax.ShapeDtypeStruct((N, INPUT_DIM), jnp.float32),
            jax.ShapeDtypeStruct((N, LATENT_DIM), jnp.float32),
        ],
        compiler_params=pltpu.CompilerParams(
            dimension_semantics=("arbitrary",),
        ),
    )(x, W_enc, b_enc.reshape(1, LATENT_DIM),
      W_dec.astype(jnp.bfloat16), b_dec.reshape(1, INPUT_DIM))
    return recon, sparse


# speedup vs baseline: 27.9087x; 1.1637x over previous
import functools

import jax
import jax.numpy as jnp
from jax.experimental import pallas as pl
from jax.experimental.pallas import tpu as pltpu

N = 8192
INPUT_DIM = 1024
LATENT_DIM = 4096
K = 32
BLOCK_ROWS = 256
N_ITERS = 22


def _body(x_ref, we_ref, be_ref, wd_ref, bd_ref, recon_ref, sparse_ref):
    lat = jnp.dot(x_ref[:], we_ref[:], preferred_element_type=jnp.float32)
    lat = lat + be_ref[:]

    lo = jnp.min(lat, axis=1, keepdims=True)
    hi = jnp.max(lat, axis=1, keepdims=True)

    for _ in range(N_ITERS):
        mid = 0.5 * (lo + hi)
        cnt = jnp.sum((lat >= mid).astype(jnp.float32), axis=1, keepdims=True)
        ge = cnt >= K
        lo, hi = jnp.where(ge, mid, lo), jnp.where(ge, hi, mid)

    sparse = jnp.where(lat >= lo, lat, 0.0)
    sparse_ref[:] = sparse
    recon = jnp.dot(sparse.astype(jnp.bfloat16), wd_ref[:],
                    preferred_element_type=jnp.float32)
    recon_ref[:] = recon + bd_ref[:]


@jax.jit
def kernel(x, W_enc, b_enc, W_dec, b_dec):
    grid = (N // BLOCK_ROWS,)
    recon, sparse = pl.pallas_call(
        _body,
        grid=grid,
        in_specs=[
            pl.BlockSpec((BLOCK_ROWS, INPUT_DIM), lambda i: (i, 0)),
            pl.BlockSpec((INPUT_DIM, LATENT_DIM), lambda i: (0, 0)),
            pl.BlockSpec((1, LATENT_DIM), lambda i: (0, 0)),
            pl.BlockSpec((LATENT_DIM, INPUT_DIM), lambda i: (0, 0)),
            pl.BlockSpec((1, INPUT_DIM), lambda i: (0, 0)),
        ],
        out_specs=[
            pl.BlockSpec((BLOCK_ROWS, INPUT_DIM), lambda i: (i, 0)),
            pl.BlockSpec((BLOCK_ROWS, LATENT_DIM), lambda i: (i, 0)),
        ],
        out_shape=[
            jax.ShapeDtypeStruct((N, INPUT_DIM), jnp.float32),
            jax.ShapeDtypeStruct((N, LATENT_DIM), jnp.float32),
        ],
        compiler_params=pltpu.CompilerParams(
            dimension_semantics=("arbitrary",),
        ),
    )(x, W_enc, b_enc.reshape(1, LATENT_DIM),
      W_dec.astype(jnp.bfloat16), b_dec.reshape(1, INPUT_DIM))
    return recon, sparse


# speedup vs baseline: 30.1398x; 1.0799x over previous
import functools

import jax
import jax.numpy as jnp
from jax.experimental import pallas as pl
from jax.experimental.pallas import tpu as pltpu

N = 8192
INPUT_DIM = 1024
LATENT_DIM = 4096
K = 32
BLOCK_ROWS = 512
N_ITERS = 20


def _body(x_ref, we_ref, be_ref, wd_ref, bd_ref, recon_ref, sparse_ref):
    lat = jnp.dot(x_ref[:], we_ref[:], preferred_element_type=jnp.float32)
    lat = lat + be_ref[:]

    lo = jnp.min(lat, axis=1, keepdims=True)
    hi = jnp.max(lat, axis=1, keepdims=True)

    for _ in range(N_ITERS):
        mid = 0.5 * (lo + hi)
        cnt = jnp.sum((lat >= mid).astype(jnp.float32), axis=1, keepdims=True)
        ge = cnt >= K
        lo, hi = jnp.where(ge, mid, lo), jnp.where(ge, hi, mid)

    sparse = jnp.where(lat >= lo, lat, 0.0)
    sparse_ref[:] = sparse
    recon = jnp.dot(sparse.astype(jnp.bfloat16), wd_ref[:],
                    preferred_element_type=jnp.float32)
    recon_ref[:] = recon + bd_ref[:]


@jax.jit
def kernel(x, W_enc, b_enc, W_dec, b_dec):
    grid = (N // BLOCK_ROWS,)
    recon, sparse = pl.pallas_call(
        _body,
        grid=grid,
        in_specs=[
            pl.BlockSpec((BLOCK_ROWS, INPUT_DIM), lambda i: (i, 0)),
            pl.BlockSpec((INPUT_DIM, LATENT_DIM), lambda i: (0, 0)),
            pl.BlockSpec((1, LATENT_DIM), lambda i: (0, 0)),
            pl.BlockSpec((LATENT_DIM, INPUT_DIM), lambda i: (0, 0)),
            pl.BlockSpec((1, INPUT_DIM), lambda i: (0, 0)),
        ],
        out_specs=[
            pl.BlockSpec((BLOCK_ROWS, INPUT_DIM), lambda i: (i, 0)),
            pl.BlockSpec((BLOCK_ROWS, LATENT_DIM), lambda i: (i, 0)),
        ],
        out_shape=[
            jax.ShapeDtypeStruct((N, INPUT_DIM), jnp.float32),
            jax.ShapeDtypeStruct((N, LATENT_DIM), jnp.float32),
        ],
        compiler_params=pltpu.CompilerParams(
            dimension_semantics=("arbitrary",),
        ),
    )(x, W_enc, b_enc.reshape(1, LATENT_DIM),
      W_dec.astype(jnp.bfloat16), b_dec.reshape(1, INPUT_DIM))
    return recon, sparse
